# baseline (speedup 1.0000x reference)
import functools

import jax
import jax.numpy as jnp
from jax import lax
from jax.experimental import pallas as pl
from jax.experimental.pallas import tpu as pltpu
from jax.experimental.pallas import tpu_sc as plsc

IN = 4096
HID = 4096
B = 64
RB = 64
DT = 0.1
NNZ = 167772

NC = 2
NS = 16
NW = NC * NS
SUBROW = 128
KSUB = 3
CHUNK = KSUB * SUBROW
NCHUNK = 14
PER_W = NCHUNK * CHUNK
NNZ_PAD = NW * PER_W
NCHUNK_ST = NCHUNK + 1
ROWS_PER_TILE = HID // NS
LANES = 16

_MESH = plsc.VectorSubcoreMesh(core_axis_name="c", subcore_axis_name="s")


def _zero_contrib(contrib):
    zero16 = jnp.zeros((LANES,), jnp.float32)

    def zrow(i, _):
        for t in range(B // LANES):
            contrib[i, pl.ds(LANES * t, LANES)] = zero16
        return 0

    lax.fori_loop(0, ROWS_PER_TILE, zrow, 0)


def _accumulate(src_hbm, rows_hbm, cols_hbm, vals_hbm, acc, wid,
                rows_v, cols_v, vals_v, cbufs, gsems, ssems):
    pltpu.sync_copy(rows_hbm.at[wid], rows_v)
    pltpu.sync_copy(cols_hbm.at[wid], cols_v)
    pltpu.sync_copy(vals_hbm.at[wid], vals_v)

    def g_start(j, b):
        return pltpu.async_copy(src_hbm.at[rows_v.at[j]], cbufs[b], gsems[b])

    def s_start(j, b):
        return pltpu.async_copy(cbufs[b], acc.at[cols_v.at[j]], ssems[b],
                                add=True)

    def scale(j, b):
        cb = cbufs[b]

        def sgroup(g, _):
            vv = vals_v[j, pl.ds(LANES * g, LANES)]
            base_r = LANES * g
            for l in range(LANES):
                v = vv[l]
                for t in range(B // LANES):
                    sl = pl.ds(LANES * t, LANES)
                    cb[base_r + l, sl] = cb[base_r + l, sl] * v
            return 0

        lax.fori_loop(0, CHUNK // LANES, sgroup, 0)

    gd = [None] * (NCHUNK + 1)
    sd = [None] * NCHUNK
    gd[0] = g_start(0, 0)
    for j in range(NCHUNK):
        b, bn = j % 3, (j + 1) % 3
        if j >= 2:
            sd[j - 2].wait()
        gd[j + 1] = g_start(j + 1, bn)
        gd[j].wait()
        scale(j, b)
        sd[j] = s_start(j, b)
    gd[NCHUNK].wait()
    sd[NCHUNK - 2].wait()
    sd[NCHUNK - 1].wait()


def _spmm_sc_kernel(n_mats):
    def body(*refs):
        srcs = refs[0:n_mats]
        coo = refs[n_mats:4 * n_mats]
        outs = refs[4 * n_mats:5 * n_mats]
        accs = refs[5 * n_mats:6 * n_mats]
        rest = refs[6 * n_mats:]
        rows_v, cols_v, vals_v = rest[0:3]
        cbufs = rest[3:6]
        gsems = rest[6:9]
        ssems = rest[9:12]

        cid = lax.axis_index("c")
        sid = lax.axis_index("s")
        wid = sid * NC + cid
        base = sid * ROWS_PER_TILE

        _zero_contrib(cbufs[2])
        for m in range(n_mats):
            pltpu.sync_copy(cbufs[2].at[pl.ds(0, ROWS_PER_TILE)],
                            accs[m].at[pl.ds(base, ROWS_PER_TILE)])
        plsc.subcore_barrier()

        for m in range(n_mats):
            _accumulate(srcs[m], coo[3 * m], coo[3 * m + 1], coo[3 * m + 2],
                        accs[m], wid, rows_v, cols_v, vals_v,
                        cbufs, gsems, ssems)
        plsc.subcore_barrier()

        for m in range(n_mats):
            pltpu.sync_copy(accs[m].at[pl.ds(base, ROWS_PER_TILE)],
                            outs[m].at[cid, pl.ds(base, ROWS_PER_TILE)])

    out_type = tuple(jax.ShapeDtypeStruct((NC, HID, B), jnp.float32)
                     for _ in range(n_mats))
    scratch = (
        [pltpu.MemorySpace.VMEM_SHARED((HID, B), jnp.float32) for _ in range(n_mats)]
        + [pltpu.VMEM((NCHUNK_ST, CHUNK), jnp.int32),
           pltpu.VMEM((NCHUNK_ST, CHUNK), jnp.int32),
           pltpu.VMEM((NCHUNK_ST, CHUNK), jnp.float32)]
        + [pltpu.VMEM((CHUNK, B), jnp.float32) for _ in range(3)]
        + [pltpu.SemaphoreType.DMA for _ in range(6)]
    )
    return pl.kernel(body, out_type=out_type, mesh=_MESH, scratch_types=scratch,
                     compiler_params=pltpu.CompilerParams(use_tc_tiling_on_sc=False))


_spmm2 = _spmm_sc_kernel(2)
_spmm1 = _spmm_sc_kernel(1)


def _fuse_body(xT, rW, rb, yW, yR, wb, rbias, hT, gT, tauT, out):
    rg = lax.dot_general(rW[...], xT[...], (((0,), (0,)), ((), ())),
                         preferred_element_type=jnp.float32)
    rg = jax.nn.sigmoid(rg + rb[...])
    mask = jnp.reshape(jnp.broadcast_to(rg[:, None, :], (RB, HID // RB, B)),
                       (HID, B))
    sensory = (yW[0] + yW[1] + wb[...]) * mask
    rec = yR[0] + yR[1] + rbias[...]
    target = jnp.tanh(sensory + rec)
    h = hT[...]
    out[...] = h + gT[...] * (target - h) * (DT / tauT[...])


_fuse = pl.pallas_call(
    _fuse_body,
    out_shape=jax.ShapeDtypeStruct((HID, B), jnp.float32),
)


def _combine_body(p, pb, out):
    out[...] = p[0] + p[1] + pb[...]


_combine = pl.pallas_call(
    _combine_body,
    out_shape=jax.ShapeDtypeStruct((HID, B), jnp.float32),
)


def _coo(rows, cols, vals):
    pad = NNZ_PAD - NNZ

    def prep(a):
        a = jnp.pad(a, (0, pad)).reshape(NW, NCHUNK, CHUNK)
        extra = jnp.zeros((NW, 1, CHUNK), a.dtype)
        return jnp.concatenate([a, extra], axis=1)

    return (prep(rows.astype(jnp.int32)), prep(cols.astype(jnp.int32)),
            prep(vals))


def kernel(x, h_prev, gate, W_rows, W_cols, W_vals, W_bias,
           R_rows, R_cols, R_vals, R_bias, P_rows, P_cols, P_vals, P_bias,
           router_W, router_b, tau):
    xT = x.T
    hT = h_prev.T
    gT = gate.T
    Wr, Wc, Wv = _coo(W_rows, W_cols, W_vals)
    Rr, Rc, Rv = _coo(R_rows, R_cols, R_vals)
    Pr, Pc, Pv = _coo(P_rows, P_cols, P_vals)

    yW, yR = _spmm2(xT, hT, Wr, Wc, Wv, Rr, Rc, Rv)
    h_newT = _fuse(xT, router_W, router_b.reshape(RB, 1), yW, yR,
                   W_bias.reshape(HID, 1), R_bias.reshape(HID, 1),
                   hT, gT, tau.reshape(HID, 1))
    (p,) = _spmm1(h_newT, Pr, Pc, Pv)
    predT = _combine(p, P_bias.reshape(HID, 1))
    return (h_newT.T, predT.T)


# speedup vs baseline: 1.6041x; 1.6041x over previous
import functools

import jax
import jax.numpy as jnp
from jax import lax
from jax.experimental import pallas as pl
from jax.experimental.pallas import tpu as pltpu
from jax.experimental.pallas import tpu_sc as plsc

IN = 4096
HID = 4096
B = 64
RB = 64
DT = 0.1
NNZ = 167772

NC = 2
NS = 16
NW = NC * NS
SUBROW = 128
KSUB = 3
CHUNK = KSUB * SUBROW
NCHUNK = 14
PER_W = NCHUNK * CHUNK
NNZ_PAD = NW * PER_W
NCHUNK_ST = NCHUNK + 1
ROWS_PER_TILE = HID // NS
LANES = 16

_MESH = plsc.VectorSubcoreMesh(core_axis_name="c", subcore_axis_name="s")


def _zero_contrib(contrib):
    zero16 = jnp.zeros((LANES,), jnp.float32)

    def zrow(i, _):
        for t in range(B // LANES):
            contrib[i, pl.ds(LANES * t, LANES)] = zero16
        return 0

    lax.fori_loop(0, ROWS_PER_TILE, zrow, 0)


def _accumulate(src_hbm, rows_hbm, cols_hbm, vals_hbm, acc, wid,
                rows_v, cols_v, vals_v, cbufs, gsems, ssems):
    pltpu.sync_copy(rows_hbm.at[wid], rows_v)
    pltpu.sync_copy(cols_hbm.at[wid], cols_v)
    pltpu.sync_copy(vals_hbm.at[wid], vals_v)

    def g_start(j, b):
        return pltpu.async_copy(src_hbm.at[rows_v.at[j]], cbufs[b], gsems[b])

    def s_start(j, b):
        return pltpu.async_copy(cbufs[b], acc.at[cols_v.at[j]], ssems[b],
                                add=True)

    def scale(j, b):
        cb = cbufs[b]

        def sgroup(g, _):
            vv = vals_v[j, pl.ds(LANES * g, LANES)]
            base_r = LANES * g
            for l in range(LANES):
                v = vv[l]
                for t in range(B // LANES):
                    sl = pl.ds(LANES * t, LANES)
                    cb[base_r + l, sl] = cb[base_r + l, sl] * v
            return 0

        lax.fori_loop(0, CHUNK // LANES, sgroup, 0)

    def chunk_body(j, _):
        g_start(j, 0).wait()
        scale(j, 0)
        pltpu.sync_copy(cbufs[0], acc.at[cols_v.at[j]], add=True)
        return 0

    lax.fori_loop(0, NCHUNK, chunk_body, 0)
    del s_start


def _spmm_sc_kernel(n_mats):
    def body(*refs):
        srcs = refs[0:n_mats]
        coo = refs[n_mats:4 * n_mats]
        outs = refs[4 * n_mats:5 * n_mats]
        accs = refs[5 * n_mats:6 * n_mats]
        rest = refs[6 * n_mats:]
        rows_v, cols_v, vals_v = rest[0:3]
        cbufs = rest[3:6]
        gsems = rest[6:9]
        ssems = rest[9:12]

        cid = lax.axis_index("c")
        sid = lax.axis_index("s")
        wid = sid * NC + cid
        base = sid * ROWS_PER_TILE

        _zero_contrib(cbufs[2])
        for m in range(n_mats):
            pltpu.sync_copy(cbufs[2].at[pl.ds(0, ROWS_PER_TILE)],
                            accs[m].at[pl.ds(base, ROWS_PER_TILE)])
        plsc.subcore_barrier()

        for m in range(n_mats):
            _accumulate(srcs[m], coo[3 * m], coo[3 * m + 1], coo[3 * m + 2],
                        accs[m], wid, rows_v, cols_v, vals_v,
                        cbufs, gsems, ssems)
        plsc.subcore_barrier()

        for m in range(n_mats):
            pltpu.sync_copy(accs[m].at[pl.ds(base, ROWS_PER_TILE)],
                            outs[m].at[cid, pl.ds(base, ROWS_PER_TILE)])

    out_type = tuple(jax.ShapeDtypeStruct((NC, HID, B), jnp.float32)
                     for _ in range(n_mats))
    scratch = (
        [pltpu.MemorySpace.VMEM_SHARED((HID, B), jnp.float32) for _ in range(n_mats)]
        + [pltpu.VMEM((NCHUNK_ST, CHUNK), jnp.int32),
           pltpu.VMEM((NCHUNK_ST, CHUNK), jnp.int32),
           pltpu.VMEM((NCHUNK_ST, CHUNK), jnp.float32)]
        + [pltpu.VMEM((CHUNK, B), jnp.float32) for _ in range(3)]
        + [pltpu.SemaphoreType.DMA for _ in range(6)]
    )
    return pl.kernel(body, out_type=out_type, mesh=_MESH, scratch_types=scratch,
                     compiler_params=pltpu.CompilerParams(use_tc_tiling_on_sc=False))


_spmm2 = _spmm_sc_kernel(2)
_spmm1 = _spmm_sc_kernel(1)


def _fuse_body(xT, rW, rb, yW, yR, wb, rbias, hT, gT, tauT, out):
    rg = lax.dot_general(rW[...], xT[...], (((0,), (0,)), ((), ())),
                         preferred_element_type=jnp.float32)
    rg = jax.nn.sigmoid(rg + rb[...])
    mask = jnp.reshape(jnp.broadcast_to(rg[:, None, :], (RB, HID // RB, B)),
                       (HID, B))
    sensory = (yW[0] + yW[1] + wb[...]) * mask
    rec = yR[0] + yR[1] + rbias[...]
    target = jnp.tanh(sensory + rec)
    h = hT[...]
    out[...] = h + gT[...] * (target - h) * (DT / tauT[...])


_fuse = pl.pallas_call(
    _fuse_body,
    out_shape=jax.ShapeDtypeStruct((HID, B), jnp.float32),
)


def _combine_body(p, pb, out):
    out[...] = p[0] + p[1] + pb[...]


_combine = pl.pallas_call(
    _combine_body,
    out_shape=jax.ShapeDtypeStruct((HID, B), jnp.float32),
)


def _coo(rows, cols, vals):
    pad = NNZ_PAD - NNZ

    def prep(a):
        a = jnp.pad(a, (0, pad)).reshape(NW, NCHUNK, CHUNK)
        extra = jnp.zeros((NW, 1, CHUNK), a.dtype)
        return jnp.concatenate([a, extra], axis=1)

    return (prep(rows.astype(jnp.int32)), prep(cols.astype(jnp.int32)),
            prep(vals))


def kernel(x, h_prev, gate, W_rows, W_cols, W_vals, W_bias,
           R_rows, R_cols, R_vals, R_bias, P_rows, P_cols, P_vals, P_bias,
           router_W, router_b, tau):
    xT = x.T
    hT = h_prev.T
    gT = gate.T
    Wr, Wc, Wv = _coo(W_rows, W_cols, W_vals)
    Rr, Rc, Rv = _coo(R_rows, R_cols, R_vals)
    Pr, Pc, Pv = _coo(P_rows, P_cols, P_vals)

    yW, yR = _spmm2(xT, hT, Wr, Wc, Wv, Rr, Rc, Rv)
    h_newT = _fuse(xT, router_W, router_b.reshape(RB, 1), yW, yR,
                   W_bias.reshape(HID, 1), R_bias.reshape(HID, 1),
                   hT, gT, tau.reshape(HID, 1))
    (p,) = _spmm1(h_newT, Pr, Pc, Pv)
    predT = _combine(p, P_bias.reshape(HID, 1))
    return (h_newT.T, predT.T)


# speedup vs baseline: 1.8110x; 1.1290x over previous
import functools

import jax
import jax.numpy as jnp
from jax import lax
from jax.experimental import pallas as pl
from jax.experimental.pallas import tpu as pltpu
from jax.experimental.pallas import tpu_sc as plsc

IN = 4096
HID = 4096
B = 64
RB = 64
DT = 0.1
NNZ = 167772

NC = 2
NS = 16
NW = NC * NS
CHUNK = 128
NCHUNK = 42
PER_W = NCHUNK * CHUNK
NNZ_PAD = NW * PER_W
NCHUNK_ST = NCHUNK + 1
ROWS_PER_TILE = HID // NS
LANES = 16

_MESH = plsc.VectorSubcoreMesh(core_axis_name="c", subcore_axis_name="s")


def _zero_contrib(contrib):
    zero16 = jnp.zeros((LANES,), jnp.float32)

    def zrow(i, _):
        for t in range(B // LANES):
            contrib[i, pl.ds(LANES * t, LANES)] = zero16
        return 0

    lax.fori_loop(0, CHUNK, zrow, 0)


def _accumulate(src_hbm, rows_hbm, cols_hbm, vals_hbm, acc, wid,
                rows_v, cols_v, vals_v, cbufs, gsems, ssems):
    pltpu.sync_copy(rows_hbm.at[wid], rows_v)
    pltpu.sync_copy(cols_hbm.at[wid], cols_v)
    pltpu.sync_copy(vals_hbm.at[wid], vals_v)

    def g_start(j, b):
        return pltpu.async_copy(src_hbm.at[rows_v.at[j]], cbufs[b], gsems[b])

    def s_start(j, b):
        return pltpu.async_copy(cbufs[b], acc.at[cols_v.at[j]], ssems[b],
                                add=True)

    def scale(j, b):
        cb = cbufs[b]

        def sgroup(g, _):
            vv = vals_v[j, pl.ds(LANES * g, LANES)]
            base_r = LANES * g
            for l in range(LANES):
                v = vv[l]
                for t in range(B // LANES):
                    sl = pl.ds(LANES * t, LANES)
                    cb[base_r + l, sl] = cb[base_r + l, sl] * v
            return 0

        lax.fori_loop(0, CHUNK // LANES, sgroup, 0)

    def group_body(s, _):
        j = 3 * s
        g0 = g_start(j, 0)
        g1 = g_start(j + 1, 1)
        g0.wait()
        scale(j, 0)
        s0 = s_start(j, 0)
        g2 = g_start(j + 2, 2)
        g1.wait()
        scale(j + 1, 1)
        s1 = s_start(j + 1, 1)
        g2.wait()
        scale(j + 2, 2)
        s2 = s_start(j + 2, 2)
        s0.wait()
        s1.wait()
        s2.wait()
        return 0

    lax.fori_loop(0, NCHUNK // 3, group_body, 0)


def _spmm_sc_kernel(n_mats):
    def body(*refs):
        srcs = refs[0:n_mats]
        coo = refs[n_mats:4 * n_mats]
        outs = refs[4 * n_mats:5 * n_mats]
        accs = refs[5 * n_mats:6 * n_mats]
        rest = refs[6 * n_mats:]
        rows_v, cols_v, vals_v = rest[0:3]
        cbufs = rest[3:6]
        gsems = rest[6:9]
        ssems = rest[9:12]

        cid = lax.axis_index("c")
        sid = lax.axis_index("s")
        wid = sid * NC + cid
        base = sid * ROWS_PER_TILE

        _zero_contrib(cbufs[2])
        for m in range(n_mats):
            for h in range(ROWS_PER_TILE // CHUNK):
                pltpu.sync_copy(
                    cbufs[2], accs[m].at[pl.ds(base + h * CHUNK, CHUNK)])
        plsc.subcore_barrier()

        for m in range(n_mats):
            _accumulate(srcs[m], coo[3 * m], coo[3 * m + 1], coo[3 * m + 2],
                        accs[m], wid, rows_v, cols_v, vals_v,
                        cbufs, gsems, ssems)
        plsc.subcore_barrier()

        for m in range(n_mats):
            pltpu.sync_copy(accs[m].at[pl.ds(base, ROWS_PER_TILE)],
                            outs[m].at[cid, pl.ds(base, ROWS_PER_TILE)])

    out_type = tuple(jax.ShapeDtypeStruct((NC, HID, B), jnp.float32)
                     for _ in range(n_mats))
    scratch = (
        [pltpu.MemorySpace.VMEM_SHARED((HID, B), jnp.float32) for _ in range(n_mats)]
        + [pltpu.VMEM((NCHUNK_ST, CHUNK), jnp.int32),
           pltpu.VMEM((NCHUNK_ST, CHUNK), jnp.int32),
           pltpu.VMEM((NCHUNK_ST, CHUNK), jnp.float32)]
        + [pltpu.VMEM((CHUNK, B), jnp.float32) for _ in range(3)]
        + [pltpu.SemaphoreType.DMA for _ in range(6)]
    )
    return pl.kernel(body, out_type=out_type, mesh=_MESH, scratch_types=scratch,
                     compiler_params=pltpu.CompilerParams(use_tc_tiling_on_sc=False))


_spmm2 = _spmm_sc_kernel(2)
_spmm1 = _spmm_sc_kernel(1)


def _fuse_body(xT, rW, rb, yW, yR, wb, rbias, hT, gT, tauT, out):
    rg = lax.dot_general(rW[...], xT[...], (((0,), (0,)), ((), ())),
                         preferred_element_type=jnp.float32)
    rg = jax.nn.sigmoid(rg + rb[...])
    mask = jnp.reshape(jnp.broadcast_to(rg[:, None, :], (RB, HID // RB, B)),
                       (HID, B))
    sensory = (yW[0] + yW[1] + wb[...]) * mask
    rec = yR[0] + yR[1] + rbias[...]
    target = jnp.tanh(sensory + rec)
    h = hT[...]
    out[...] = h + gT[...] * (target - h) * (DT / tauT[...])


_fuse = pl.pallas_call(
    _fuse_body,
    out_shape=jax.ShapeDtypeStruct((HID, B), jnp.float32),
)


def _combine_body(p, pb, out):
    out[...] = p[0] + p[1] + pb[...]


_combine = pl.pallas_call(
    _combine_body,
    out_shape=jax.ShapeDtypeStruct((HID, B), jnp.float32),
)


def _coo(rows, cols, vals):
    pad = NNZ_PAD - NNZ

    def prep(a):
        a = jnp.pad(a, (0, pad)).reshape(NW, NCHUNK, CHUNK)
        extra = jnp.zeros((NW, 1, CHUNK), a.dtype)
        return jnp.concatenate([a, extra], axis=1)

    return (prep(rows.astype(jnp.int32)), prep(cols.astype(jnp.int32)),
            prep(vals))


def kernel(x, h_prev, gate, W_rows, W_cols, W_vals, W_bias,
           R_rows, R_cols, R_vals, R_bias, P_rows, P_cols, P_vals, P_bias,
           router_W, router_b, tau):
    xT = x.T
    hT = h_prev.T
    gT = gate.T
    Wr, Wc, Wv = _coo(W_rows, W_cols, W_vals)
    Rr, Rc, Rv = _coo(R_rows, R_cols, R_vals)
    Pr, Pc, Pv = _coo(P_rows, P_cols, P_vals)

    yW, yR = _spmm2(xT, hT, Wr, Wc, Wv, Rr, Rc, Rv)
    h_newT = _fuse(xT, router_W, router_b.reshape(RB, 1), yW, yR,
                   W_bias.reshape(HID, 1), R_bias.reshape(HID, 1),
                   hT, gT, tau.reshape(HID, 1))
    (p,) = _spmm1(h_newT, Pr, Pc, Pv)
    predT = _combine(p, P_bias.reshape(HID, 1))
    return (h_newT.T, predT.T)


# speedup vs baseline: 2.3658x; 1.3063x over previous
import functools

import jax
import jax.numpy as jnp
from jax import lax
from jax.experimental import pallas as pl
from jax.experimental.pallas import tpu as pltpu
from jax.experimental.pallas import tpu_sc as plsc

IN = 4096
HID = 4096
B = 64
RB = 64
DT = 0.1
NNZ = 167772

NC = 2
NS = 16
NW = NC * NS
CHUNK = 128
NCHUNK = 41
PER_W = NCHUNK * CHUNK
NNZ_PAD = NW * PER_W
NCHUNK_ST = NCHUNK + 1
ROWS_PER_TILE = HID // NS
LANES = 16

_MESH = plsc.VectorSubcoreMesh(core_axis_name="c", subcore_axis_name="s")


def _zero_contrib(contrib):
    zero16 = jnp.zeros((LANES,), jnp.float32)

    def zrow(i, _):
        for t in range(B // LANES):
            contrib[i, pl.ds(LANES * t, LANES)] = zero16
        return 0

    lax.fori_loop(0, CHUNK, zrow, 0)


def _accumulate(src_hbm, rows_hbm, cols_hbm, vals_hbm, acc, wid,
                rows_v, cols_v, vals_v, cbufs, gsems, ssems):
    pltpu.sync_copy(rows_hbm.at[wid], rows_v)
    pltpu.sync_copy(cols_hbm.at[wid], cols_v)
    pltpu.sync_copy(vals_hbm.at[wid], vals_v)

    def g_start(j, b):
        return pltpu.async_copy(src_hbm.at[rows_v.at[j]], cbufs[b], gsems[b])

    def s_start(j, b):
        return pltpu.async_copy(cbufs[b], acc.at[cols_v.at[j]], ssems[b],
                                add=True)

    def scale(j, b):
        cb = cbufs[b]

        def sgroup(g, _):
            vv = vals_v[j, pl.ds(LANES * g, LANES)]
            base_r = LANES * g
            for l in range(LANES):
                v = vv[l]
                for t in range(B // LANES):
                    sl = pl.ds(LANES * t, LANES)
                    cb[base_r + l, sl] = cb[base_r + l, sl] * v
            return 0

        lax.fori_loop(0, CHUNK // LANES, sgroup, 0)

    def chunk_body(j, _):
        g_start(j, 0).wait()
        scale(j, 0)
        pltpu.sync_copy(cbufs[0], acc.at[cols_v.at[j]], add=True)
        return 0

    lax.fori_loop(0, NCHUNK, chunk_body, 0)
    del s_start


def _spmm_sc_kernel(n_mats):
    def body(*refs):
        srcs = refs[0:n_mats]
        coo = refs[n_mats:4 * n_mats]
        outs = refs[4 * n_mats:5 * n_mats]
        accs = refs[5 * n_mats:6 * n_mats]
        rest = refs[6 * n_mats:]
        rows_v, cols_v, vals_v = rest[0:3]
        cbufs = rest[3:6]
        gsems = rest[6:9]
        ssems = rest[9:12]

        cid = lax.axis_index("c")
        sid = lax.axis_index("s")
        wid = sid * NC + cid
        base = sid * ROWS_PER_TILE

        _zero_contrib(cbufs[2])
        for m in range(n_mats):
            for h in range(ROWS_PER_TILE // CHUNK):
                pltpu.sync_copy(
                    cbufs[2], accs[m].at[pl.ds(base + h * CHUNK, CHUNK)])
        plsc.subcore_barrier()

        for m in range(n_mats):
            _accumulate(srcs[m], coo[3 * m], coo[3 * m + 1], coo[3 * m + 2],
                        accs[m], wid, rows_v, cols_v, vals_v,
                        cbufs, gsems, ssems)
        plsc.subcore_barrier()

        for m in range(n_mats):
            pltpu.sync_copy(accs[m].at[pl.ds(base, ROWS_PER_TILE)],
                            outs[m].at[cid, pl.ds(base, ROWS_PER_TILE)])

    out_type = tuple(jax.ShapeDtypeStruct((NC, HID, B), jnp.float32)
                     for _ in range(n_mats))
    scratch = (
        [pltpu.MemorySpace.VMEM_SHARED((HID, B), jnp.float32) for _ in range(n_mats)]
        + [pltpu.VMEM((NCHUNK_ST, CHUNK), jnp.int32),
           pltpu.VMEM((NCHUNK_ST, CHUNK), jnp.int32),
           pltpu.VMEM((NCHUNK_ST, CHUNK), jnp.float32)]
        + [pltpu.VMEM((CHUNK, B), jnp.float32) for _ in range(3)]
        + [pltpu.SemaphoreType.DMA for _ in range(6)]
    )
    return pl.kernel(body, out_type=out_type, mesh=_MESH, scratch_types=scratch,
                     compiler_params=pltpu.CompilerParams(use_tc_tiling_on_sc=False))


_spmm2 = _spmm_sc_kernel(2)
_spmm1 = _spmm_sc_kernel(1)


def _fuse_body(xT, rW, rb, yW, yR, wb, rbias, hT, gT, tauT, out):
    rg = lax.dot_general(rW[...], xT[...], (((0,), (0,)), ((), ())),
                         preferred_element_type=jnp.float32)
    rg = jax.nn.sigmoid(rg + rb[...])
    mask = jnp.reshape(jnp.broadcast_to(rg[:, None, :], (RB, HID // RB, B)),
                       (HID, B))
    sensory = (yW[0] + yW[1] + wb[...]) * mask
    rec = yR[0] + yR[1] + rbias[...]
    target = jnp.tanh(sensory + rec)
    h = hT[...]
    out[...] = h + gT[...] * (target - h) * (DT / tauT[...])


_fuse = pl.pallas_call(
    _fuse_body,
    out_shape=jax.ShapeDtypeStruct((HID, B), jnp.float32),
)


def _combine_body(p, pb, out):
    out[...] = p[0] + p[1] + pb[...]


_combine = pl.pallas_call(
    _combine_body,
    out_shape=jax.ShapeDtypeStruct((HID, B), jnp.float32),
)


def _coo(rows, cols, vals):
    pad = NNZ_PAD - NNZ

    def prep(a):
        a = jnp.pad(a, (0, pad)).reshape(NW, NCHUNK, CHUNK)
        extra = jnp.zeros((NW, 1, CHUNK), a.dtype)
        return jnp.concatenate([a, extra], axis=1)

    return (prep(rows.astype(jnp.int32)), prep(cols.astype(jnp.int32)),
            prep(vals))


def kernel(x, h_prev, gate, W_rows, W_cols, W_vals, W_bias,
           R_rows, R_cols, R_vals, R_bias, P_rows, P_cols, P_vals, P_bias,
           router_W, router_b, tau):
    xT = x.T
    hT = h_prev.T
    gT = gate.T
    Wr, Wc, Wv = _coo(W_rows, W_cols, W_vals)
    Rr, Rc, Rv = _coo(R_rows, R_cols, R_vals)
    Pr, Pc, Pv = _coo(P_rows, P_cols, P_vals)

    yW, yR = _spmm2(xT, hT, Wr, Wc, Wv, Rr, Rc, Rv)
    h_newT = _fuse(xT, router_W, router_b.reshape(RB, 1), yW, yR,
                   W_bias.reshape(HID, 1), R_bias.reshape(HID, 1),
                   hT, gT, tau.reshape(HID, 1))
    (p,) = _spmm1(h_newT, Pr, Pc, Pv)
    predT = _combine(p, P_bias.reshape(HID, 1))
    return (h_newT.T, predT.T)


# speedup vs baseline: 2.3660x; 1.0001x over previous
import jax
import jax.numpy as jnp
from jax import lax
from jax.experimental import pallas as pl
from jax.experimental.pallas import tpu as pltpu
from jax.experimental.pallas import tpu_sc as plsc

IN = 4096
HID = 4096
B = 64
RB = 64
DT = 0.1
NNZ = 167772

NC = 2
NS = 16
NW = NC * NS
CHUNK = 128
NCHUNK = 41
PER_W = NCHUNK * CHUNK
NNZ_PAD = NW * PER_W
NCHUNK_ST = NCHUNK + 1
ROWS_PER_TILE = HID // NS
LANES = 16

_MESH = plsc.VectorSubcoreMesh(core_axis_name="c", subcore_axis_name="s")


def _zero_contrib(contrib):
    zero16 = jnp.zeros((LANES,), jnp.float32)

    def zrow(i, _):
        for t in range(B // LANES):
            contrib[i, pl.ds(LANES * t, LANES)] = zero16
        return 0

    lax.fori_loop(0, CHUNK, zrow, 0)


def _accumulate(src_hbm, rows_hbm, cols_hbm, vals_hbm, acc, wid,
                rows_v, cols_v, vals_v, cbufs, gsems, ssems):
    pltpu.sync_copy(rows_hbm.at[wid], rows_v)
    pltpu.sync_copy(cols_hbm.at[wid], cols_v)
    pltpu.sync_copy(vals_hbm.at[wid], vals_v)

    def g_start(j, b):
        return pltpu.async_copy(src_hbm.at[rows_v.at[j]], cbufs[b], gsems[b])

    def s_start(j, b):
        return pltpu.async_copy(cbufs[b], acc.at[cols_v.at[j]], ssems[b],
                                add=True)

    def scale(j, b):
        cb = cbufs[b]

        def sgroup(g, _):
            vv = vals_v[j, pl.ds(LANES * g, LANES)]
            base_r = LANES * g
            for l in range(LANES):
                v = vv[l]
                for t in range(B // LANES):
                    sl = pl.ds(LANES * t, LANES)
                    cb[base_r + l, sl] = cb[base_r + l, sl] * v
            return 0

        lax.fori_loop(0, CHUNK // LANES, sgroup, 0)

    def chunk_body(j, _):
        g_start(j, 0).wait()
        scale(j, 0)
        pltpu.sync_copy(cbufs[0], acc.at[cols_v.at[j]], add=True)
        return 0

    lax.fori_loop(0, NCHUNK, chunk_body, 0)
    del s_start


def _spmm_sc_kernel(n_mats):
    def body(*refs):
        srcs = refs[0:n_mats]
        coo = refs[n_mats:4 * n_mats]
        outs = refs[4 * n_mats:5 * n_mats]
        accs = refs[5 * n_mats:6 * n_mats]
        rest = refs[6 * n_mats:]
        rows_v, cols_v, vals_v = rest[0:3]
        cbufs = rest[3:6]
        gsems = rest[6:9]
        ssems = rest[9:12]

        cid = lax.axis_index("c")
        sid = lax.axis_index("s")
        wid = sid * NC + cid
        base = sid * ROWS_PER_TILE

        _zero_contrib(cbufs[2])
        for m in range(n_mats):
            for h in range(ROWS_PER_TILE // CHUNK):
                pltpu.sync_copy(
                    cbufs[2], accs[m].at[pl.ds(base + h * CHUNK, CHUNK)])
        plsc.subcore_barrier()

        for m in range(n_mats):
            _accumulate(srcs[m], coo[3 * m], coo[3 * m + 1], coo[3 * m + 2],
                        accs[m], wid, rows_v, cols_v, vals_v,
                        cbufs, gsems, ssems)
        plsc.subcore_barrier()

        for m in range(n_mats):
            pltpu.sync_copy(accs[m].at[pl.ds(base, ROWS_PER_TILE)],
                            outs[m].at[cid, pl.ds(base, ROWS_PER_TILE)])

    out_type = tuple(jax.ShapeDtypeStruct((NC, HID, B), jnp.float32)
                     for _ in range(n_mats))
    scratch = (
        [pltpu.MemorySpace.VMEM_SHARED((HID, B), jnp.float32) for _ in range(n_mats)]
        + [pltpu.VMEM((NCHUNK_ST, CHUNK), jnp.int32),
           pltpu.VMEM((NCHUNK_ST, CHUNK), jnp.int32),
           pltpu.VMEM((NCHUNK_ST, CHUNK), jnp.float32)]
        + [pltpu.VMEM((CHUNK, B), jnp.float32) for _ in range(3)]
        + [pltpu.SemaphoreType.DMA for _ in range(6)]
    )
    return pl.kernel(body, out_type=out_type, mesh=_MESH, scratch_types=scratch,
                     compiler_params=pltpu.CompilerParams(use_tc_tiling_on_sc=False))


_spmm2 = _spmm_sc_kernel(2)
_spmm1 = _spmm_sc_kernel(1)


def _fuse_body(xT, rW, rb, yW, yR, wb, rbias, hT, gT, tauT, out):
    rg = lax.dot_general(rW[...], xT[...], (((0,), (0,)), ((), ())),
                         preferred_element_type=jnp.float32)
    rg = jax.nn.sigmoid(rg + rb[...])
    mask = jnp.reshape(jnp.broadcast_to(rg[:, None, :], (RB, HID // RB, B)),
                       (HID, B))
    sensory = (yW[0] + yW[1] + wb[...]) * mask
    rec = yR[0] + yR[1] + rbias[...]
    target = jnp.tanh(sensory + rec)
    h = hT[...]
    out[...] = h + gT[...] * (target - h) * (DT / tauT[...])


_fuse = pl.pallas_call(
    _fuse_body,
    out_shape=jax.ShapeDtypeStruct((HID, B), jnp.float32),
)


def _combine_body(p, pb, out):
    out[...] = p[0] + p[1] + pb[...]


_combine = pl.pallas_call(
    _combine_body,
    out_shape=jax.ShapeDtypeStruct((HID, B), jnp.float32),
)


def _coo(rows, cols, vals):
    pad = NNZ_PAD - NNZ

    def prep(a):
        a = jnp.pad(a, (0, pad)).reshape(NW, NCHUNK, CHUNK)
        extra = jnp.zeros((NW, 1, CHUNK), a.dtype)
        return jnp.concatenate([a, extra], axis=1)

    return (prep(rows.astype(jnp.int32)), prep(cols.astype(jnp.int32)),
            prep(vals))


def kernel(x, h_prev, gate, W_rows, W_cols, W_vals, W_bias,
           R_rows, R_cols, R_vals, R_bias, P_rows, P_cols, P_vals, P_bias,
           router_W, router_b, tau):
    xT = x.T
    hT = h_prev.T
    gT = gate.T
    Wr, Wc, Wv = _coo(W_rows, W_cols, W_vals)
    Rr, Rc, Rv = _coo(R_rows, R_cols, R_vals)
    Pr, Pc, Pv = _coo(P_rows, P_cols, P_vals)

    yW, yR = _spmm2(xT, hT, Wr, Wc, Wv, Rr, Rc, Rv)
    h_newT = _fuse(xT, router_W, router_b.reshape(RB, 1), yW, yR,
                   W_bias.reshape(HID, 1), R_bias.reshape(HID, 1),
                   hT, gT, tau.reshape(HID, 1))
    (p,) = _spmm1(h_newT, Pr, Pc, Pv)
    predT = _combine(p, P_bias.reshape(HID, 1))
    return (h_newT.T, predT.T)
